# baseline (speedup 1.0000x reference)
import functools

import jax
import jax.numpy as jnp
from jax import lax
from jax.experimental import pallas as pl
from jax.experimental.pallas import tpu as pltpu
from jax.experimental.pallas import tpu_sc as plsc

N_NODES = 10000
N_PAD = 10240
E = 320000
E_PAD = 327680
H = 128
NG = 64

NC, NS = 2, 16
NW = NC * NS
CH = 128
CPT = E_PAD // (NW * CH)
EC = 64
ECT = E_PAD // (NW * EC)
IB = 32
RPT = N_PAD // NS
ZB = 160
XPT = N_PAD // NW
XCH = 32

TB = 1024
PB = 1000

_mesh = plsc.VectorSubcoreMesh(core_axis_name="c", subcore_axis_name="s")


@functools.partial(
    pl.kernel,
    out_type=(
        jax.ShapeDtypeStruct((N_PAD, H), jnp.float32),
        jax.ShapeDtypeStruct((NC, N_PAD, H), jnp.float32),
    ),
    mesh=_mesh,
    scratch_types=[
        pltpu.VMEM((XPT,), jnp.int32),
        pltpu.VMEM((XCH, H), jnp.float32),
        pltpu.VMEM((2, CH), jnp.int32),
        pltpu.VMEM((CH, H), jnp.float32),
        pltpu.VMEM_SHARED((N_PAD, H), jnp.float32),
    ],
)
def _sc_gather_deg(emb_hbm, x_hbm, rows_hbm, znode_hbm,
                   h0_hbm, deg_hbm, xv, hbuf, rowv, ones, degsh):
    c = lax.axis_index("c")
    s = lax.axis_index("s")
    wid = s * NC + c
    base = wid * CPT
    pltpu.sync_copy(znode_hbm.at[pl.ds(0, CH)], ones)

    @pl.loop(0, RPT // CH)
    def _(q):
        pltpu.sync_copy(ones, degsh.at[pl.ds(s * RPT + q * CH, CH)])

    pltpu.sync_copy(x_hbm.at[pl.ds(wid * XPT, XPT)], xv)

    @pl.loop(0, XPT // XCH)
    def _(k):
        pltpu.sync_copy(emb_hbm.at[xv.at[pl.ds(k * XCH, XCH)]], hbuf)
        pltpu.sync_copy(hbuf, h0_hbm.at[pl.ds(wid * XPT + k * XCH, XCH)])

    pltpu.sync_copy(znode_hbm.at[pl.ds(N_PAD, CH)], ones)
    plsc.subcore_barrier()

    @pl.loop(0, CPT)
    def _(j):
        pltpu.sync_copy(rows_hbm.at[base + j], rowv.at[0])
        pltpu.sync_copy(ones, degsh.at[rowv.at[0]], add=True)

    plsc.subcore_barrier()

    @pl.loop(0, RPT // CH)
    def _(q):
        pltpu.sync_copy(degsh.at[pl.ds(s * RPT + q * CH, CH)], ones)
        pltpu.sync_copy(ones, deg_hbm.at[c, pl.ds(s * RPT + q * CH, CH)])


@functools.partial(
    pl.kernel,
    out_type=jax.ShapeDtypeStruct((NC, N_PAD, H), jnp.float32),
    mesh=_mesh,
    scratch_types=[
        pltpu.VMEM((IB, EC), jnp.int32),
        pltpu.VMEM((IB, EC), jnp.int32),
        pltpu.VMEM((EC, H), jnp.float32),
        pltpu.VMEM((EC, H), jnp.float32),
        pltpu.VMEM_SHARED((N_PAD, H), jnp.float32),
        pltpu.SemaphoreType.DMA,
        pltpu.SemaphoreType.DMA,
        pltpu.SemaphoreType.DMA,
        pltpu.SemaphoreType.DMA,
    ],
)
def _sc_edge_pass(g_hbm, col_hbm, rows_hbm, znode_hbm, acc_hbm,
                  colb, rowb, buf0, buf1, accsh, sem0, sem1, sem2, sem3):
    c = lax.axis_index("c")
    s = lax.axis_index("s")
    wid = s * NC + c
    base = wid * ECT
    pltpu.sync_copy(znode_hbm.at[pl.ds(0, EC)], buf0)

    @pl.loop(0, RPT // EC)
    def _(q):
        pltpu.sync_copy(buf0, accsh.at[pl.ds(s * RPT + q * EC, EC)])

    plsc.subcore_barrier()

    @pl.loop(0, ECT // IB)
    def _(t):
        tb = base + t * IB
        pltpu.sync_copy(col_hbm.at[pl.ds(tb, IB)], colb)
        pltpu.sync_copy(rows_hbm.at[pl.ds(tb, IB)], rowb)
        pltpu.async_copy(g_hbm.at[colb.at[0]], buf0, sem0)
        pltpu.async_copy(g_hbm.at[colb.at[1]], buf1, sem1)

        @pl.loop(0, IB - 2, step=2)
        def _(j):
            pltpu.make_async_copy(g_hbm.at[colb.at[j]], buf0, sem0).wait()
            pltpu.async_copy(buf0, accsh.at[rowb.at[j]], sem2, add=True)
            pltpu.make_async_copy(g_hbm.at[colb.at[j + 1]], buf1, sem1).wait()
            pltpu.async_copy(buf1, accsh.at[rowb.at[j + 1]], sem3, add=True)
            pltpu.make_async_copy(buf0, accsh.at[rowb.at[j]], sem2).wait()
            pltpu.async_copy(g_hbm.at[colb.at[j + 2]], buf0, sem0)
            pltpu.make_async_copy(buf1, accsh.at[rowb.at[j + 1]], sem3).wait()
            pltpu.async_copy(g_hbm.at[colb.at[j + 3]], buf1, sem1)

        pltpu.make_async_copy(g_hbm.at[colb.at[IB - 2]], buf0, sem0).wait()
        pltpu.sync_copy(buf0, accsh.at[rowb.at[IB - 2]], add=True)
        pltpu.make_async_copy(g_hbm.at[colb.at[IB - 1]], buf1, sem1).wait()
        pltpu.sync_copy(buf1, accsh.at[rowb.at[IB - 1]], add=True)

    plsc.subcore_barrier()

    @pl.loop(0, RPT // (2 * EC))
    def _(q):
        r0 = s * RPT + 2 * q * EC
        r1 = r0 + EC
        pltpu.sync_copy(accsh.at[pl.ds(r0, EC)], buf0)
        pltpu.async_copy(buf0, acc_hbm.at[c, pl.ds(r0, EC)], sem0)
        pltpu.sync_copy(accsh.at[pl.ds(r1, EC)], buf1)
        pltpu.async_copy(buf1, acc_hbm.at[c, pl.ds(r1, EC)], sem1)
        pltpu.make_async_copy(buf0, acc_hbm.at[c, pl.ds(r0, EC)], sem0).wait()
        pltpu.make_async_copy(buf1, acc_hbm.at[c, pl.ds(r1, EC)], sem1).wait()


def _dis_block(deg_ref):
    d = deg_ref[0, :, 0:1] + deg_ref[1, :, 0:1] + 1.0
    return lax.rsqrt(d)


def _tc_first_body(deg_ref, h_ref, w_ref, out_ref):
    dis = _dis_block(deg_ref)
    out_ref[...] = dis * jnp.dot(h_ref[...], w_ref[...],
                                 preferred_element_type=jnp.float32)


def _tc_mid_body(deg_ref, acc_ref, g_ref, b_ref, w_ref, out_ref):
    dis = _dis_block(deg_ref)
    h = jax.nn.relu(dis * (acc_ref[0] + acc_ref[1] + g_ref[...]) + b_ref[...])
    out_ref[...] = dis * jnp.dot(h, w_ref[...],
                                 preferred_element_type=jnp.float32)


def _tc_pool_body(deg_ref, acc_ref, g_ref, b_ref, bv_ref,
                  fc1w_ref, fc1b_ref, fc2w_ref, fc2b_ref,
                  out_ref, sacc, cacc):
    pi = pl.program_id(0)

    @pl.when(pi == 0)
    def _():
        sacc[...] = jnp.zeros_like(sacc)
        cacc[...] = jnp.zeros_like(cacc)

    dis = _dis_block(deg_ref)
    h = jax.nn.relu(dis * (acc_ref[0] + acc_ref[1] + g_ref[...]) + b_ref[...])
    bv = bv_ref[0]
    ohT = (lax.broadcasted_iota(jnp.int32, (NG, PB), 0) == bv
           ).astype(jnp.float32)
    sacc[...] += jnp.dot(ohT, h, preferred_element_type=jnp.float32)
    cacc[...] += jnp.sum(ohT, axis=1, keepdims=True)

    @pl.when(pi == (N_NODES // PB) - 1)
    def _():
        mean = sacc[...] / (cacc[...] + 1e-6)
        o1 = jax.nn.relu(jnp.dot(mean, fc1w_ref[...],
                                 preferred_element_type=jnp.float32)
                         + fc1b_ref[...])
        o2 = jnp.dot(o1, fc2w_ref[...],
                     preferred_element_type=jnp.float32) + fc2b_ref[...]
        out_ref[...] = o2[:, 0:1]


def _tc_first(degacc, h0, W1):
    return pl.pallas_call(
        _tc_first_body,
        grid=(N_PAD // TB,),
        in_specs=[
            pl.BlockSpec((NC, TB, H), lambda i: (0, i, 0)),
            pl.BlockSpec((TB, H), lambda i: (i, 0)),
            pl.BlockSpec((H, H), lambda i: (0, 0)),
        ],
        out_specs=pl.BlockSpec((TB, H), lambda i: (i, 0)),
        out_shape=jax.ShapeDtypeStruct((N_PAD, H), jnp.float32),
    )(degacc, h0, W1)


def _tc_mid(degacc, acc, g, br, Wn):
    return pl.pallas_call(
        _tc_mid_body,
        grid=(N_PAD // TB,),
        in_specs=[
            pl.BlockSpec((NC, TB, H), lambda i: (0, i, 0)),
            pl.BlockSpec((NC, TB, H), lambda i: (0, i, 0)),
            pl.BlockSpec((TB, H), lambda i: (i, 0)),
            pl.BlockSpec((1, H), lambda i: (0, 0)),
            pl.BlockSpec((H, H), lambda i: (0, 0)),
        ],
        out_specs=pl.BlockSpec((TB, H), lambda i: (i, 0)),
        out_shape=jax.ShapeDtypeStruct((N_PAD, H), jnp.float32),
    )(degacc, acc, g, br, Wn)


def _tc_pool(degacc, acc, g, br, batch_r, fc1p, fc1bp, fc2p, fc2bp):
    return pl.pallas_call(
        _tc_pool_body,
        grid=(N_NODES // PB,),
        in_specs=[
            pl.BlockSpec((NC, PB, H), lambda i: (0, i, 0)),
            pl.BlockSpec((NC, PB, H), lambda i: (0, i, 0)),
            pl.BlockSpec((PB, H), lambda i: (i, 0)),
            pl.BlockSpec((1, H), lambda i: (0, 0)),
            pl.BlockSpec((1, 1, PB), lambda i: (i, 0, 0)),
            pl.BlockSpec((H, H), lambda i: (0, 0)),
            pl.BlockSpec((1, H), lambda i: (0, 0)),
            pl.BlockSpec((H, H), lambda i: (0, 0)),
            pl.BlockSpec((1, H), lambda i: (0, 0)),
        ],
        out_specs=pl.BlockSpec((NG, 1), lambda i: (0, 0)),
        out_shape=jax.ShapeDtypeStruct((NG, 1), jnp.float32),
        scratch_shapes=[
            pltpu.VMEM((NG, H), jnp.float32),
            pltpu.VMEM((NG, 1), jnp.float32),
        ],
    )(degacc, acc, g, br, batch_r, fc1p, fc1bp, fc2p, fc2bp)


@jax.jit
def _impl(x, edge_index, batch_vec, emb, W1, b1, W2, b2, W3, b3,
          fc1_w, fc1_b, fc2_w, fc2_b):
    f32 = jnp.float32
    i32 = jnp.int32
    x_pad = jnp.concatenate(
        [x.astype(i32), jnp.zeros((N_PAD - N_NODES,), i32)])
    row = edge_index[0].astype(i32)
    col = edge_index[1].astype(i32)
    row_pad = jnp.concatenate(
        [row, jnp.full((E_PAD - E,), N_NODES, i32)]).reshape(NW * CPT, CH)
    col_pad = jnp.concatenate(
        [col, jnp.zeros((E_PAD - E,), i32)]).reshape(NW * CPT, CH)
    row_pad_e = row_pad.reshape(NW * ECT, EC)
    col_pad_e = col_pad.reshape(NW * ECT, EC)
    znode = jnp.concatenate(
        [jnp.zeros((N_PAD, H), f32), jnp.ones((CH, H), f32)])
    batch_r = batch_vec.astype(i32).reshape(N_NODES // PB, 1, PB)
    b1r = b1.reshape(1, H)
    b2r = b2.reshape(1, H)
    b3r = b3.reshape(1, H)
    fc1p = jnp.zeros((H, H), f32).at[:, :32].set(fc1_w)
    fc1bp = jnp.zeros((1, H), f32).at[0, :32].set(fc1_b)
    fc2p = jnp.zeros((H, H), f32).at[:32, 0:1].set(fc2_w)
    fc2bp = jnp.full((1, H), fc2_b[0], f32)

    h0, degacc = _sc_gather_deg(emb, x_pad, row_pad, znode)
    g1 = _tc_first(degacc, h0, W1)
    acc1 = _sc_edge_pass(g1, col_pad_e, row_pad_e, znode)
    g2 = _tc_mid(degacc, acc1, g1, b1r, W2)
    acc2 = _sc_edge_pass(g2, col_pad_e, row_pad_e, znode)
    g3 = _tc_mid(degacc, acc2, g2, b2r, W3)
    acc3 = _sc_edge_pass(g3, col_pad_e, row_pad_e, znode)
    return _tc_pool(degacc, acc3, g3, b3r, batch_r, fc1p, fc1bp, fc2p, fc2bp)


def kernel(x, edge_index, batch_vec, emb, W1, b1, W2, b2, W3, b3,
           fc1_w, fc1_b, fc2_w, fc2_b):
    return _impl(x, edge_index, batch_vec, emb, W1, b1, W2, b2, W3, b3,
                 fc1_w, fc1_b, fc2_w, fc2_b)


# speedup vs baseline: 1.1805x; 1.1805x over previous
import functools

import jax
import jax.numpy as jnp
from jax import lax
from jax.experimental import pallas as pl
from jax.experimental.pallas import tpu as pltpu
from jax.experimental.pallas import tpu_sc as plsc

N_NODES = 10000
N_PAD = 10240
E = 320000
E_PAD = 327680
H = 128
NG = 64

NC, NS = 2, 16
NW = NC * NS
CH = 128
CPT = E_PAD // (NW * CH)
EC = 80
ECT = E_PAD // (NW * EC)
IB = 16
RPT = N_PAD // NS
ZB = 160
XPT = N_PAD // NW
XCH = 32

TB = 1024
PB = 1000

_mesh = plsc.VectorSubcoreMesh(core_axis_name="c", subcore_axis_name="s")


@functools.partial(
    pl.kernel,
    out_type=(
        jax.ShapeDtypeStruct((N_PAD, H), jnp.float32),
        jax.ShapeDtypeStruct((NC, N_PAD, H), jnp.float32),
    ),
    mesh=_mesh,
    scratch_types=[
        pltpu.VMEM((XPT,), jnp.int32),
        pltpu.VMEM((XCH, H), jnp.float32),
        pltpu.VMEM((2, CH), jnp.int32),
        pltpu.VMEM((CH, H), jnp.float32),
        pltpu.VMEM_SHARED((N_PAD, H), jnp.float32),
    ],
)
def _sc_gather_deg(emb_hbm, x_hbm, rows_hbm, znode_hbm,
                   h0_hbm, deg_hbm, xv, hbuf, rowv, ones, degsh):
    c = lax.axis_index("c")
    s = lax.axis_index("s")
    wid = s * NC + c
    base = wid * CPT
    pltpu.sync_copy(znode_hbm.at[pl.ds(0, CH)], ones)

    @pl.loop(0, RPT // CH)
    def _(q):
        pltpu.sync_copy(ones, degsh.at[pl.ds(s * RPT + q * CH, CH)])

    pltpu.sync_copy(x_hbm.at[pl.ds(wid * XPT, XPT)], xv)

    @pl.loop(0, XPT // XCH)
    def _(k):
        pltpu.sync_copy(emb_hbm.at[xv.at[pl.ds(k * XCH, XCH)]], hbuf)
        pltpu.sync_copy(hbuf, h0_hbm.at[pl.ds(wid * XPT + k * XCH, XCH)])

    pltpu.sync_copy(znode_hbm.at[pl.ds(N_PAD, CH)], ones)
    plsc.subcore_barrier()

    @pl.loop(0, CPT)
    def _(j):
        pltpu.sync_copy(rows_hbm.at[base + j], rowv.at[0])
        pltpu.sync_copy(ones, degsh.at[rowv.at[0]], add=True)

    plsc.subcore_barrier()

    @pl.loop(0, RPT // CH)
    def _(q):
        pltpu.sync_copy(degsh.at[pl.ds(s * RPT + q * CH, CH)], ones)
        pltpu.sync_copy(ones, deg_hbm.at[c, pl.ds(s * RPT + q * CH, CH)])


@functools.partial(
    pl.kernel,
    out_type=jax.ShapeDtypeStruct((NC, N_PAD, H), jnp.float32),
    mesh=_mesh,
    scratch_types=[
        pltpu.VMEM((IB, EC), jnp.int32),
        pltpu.VMEM((IB, EC), jnp.int32),
        pltpu.VMEM((EC, H), jnp.float32),
        pltpu.VMEM((EC, H), jnp.float32),
        pltpu.VMEM_SHARED((N_PAD, H), jnp.float32),
        pltpu.SemaphoreType.DMA,
        pltpu.SemaphoreType.DMA,
    ],
)
def _sc_edge_pass(g_hbm, col_hbm, rows_hbm, znode_hbm, acc_hbm,
                  colb, rowb, buf0, buf1, accsh, sem0, sem1):
    c = lax.axis_index("c")
    s = lax.axis_index("s")
    wid = s * NC + c
    base = wid * ECT
    pltpu.sync_copy(znode_hbm.at[pl.ds(0, EC)], buf0)

    @pl.loop(0, RPT // EC)
    def _(q):
        pltpu.sync_copy(buf0, accsh.at[pl.ds(s * RPT + q * EC, EC)])

    plsc.subcore_barrier()

    @pl.loop(0, ECT // IB)
    def _(t):
        tb = base + t * IB
        pltpu.sync_copy(col_hbm.at[pl.ds(tb, IB)], colb)
        pltpu.sync_copy(rows_hbm.at[pl.ds(tb, IB)], rowb)
        pltpu.async_copy(g_hbm.at[colb.at[0]], buf0, sem0)

        @pl.loop(0, IB, step=2)
        def _(j):
            pltpu.async_copy(g_hbm.at[colb.at[j + 1]], buf1, sem1)
            pltpu.make_async_copy(g_hbm.at[colb.at[j]], buf0, sem0).wait()
            pltpu.sync_copy(buf0, accsh.at[rowb.at[j]], add=True)

            @pl.when(j + 2 < IB)
            def _():
                pltpu.async_copy(g_hbm.at[colb.at[j + 2]], buf0, sem0)

            pltpu.make_async_copy(g_hbm.at[colb.at[j + 1]], buf1, sem1).wait()
            pltpu.sync_copy(buf1, accsh.at[rowb.at[j + 1]], add=True)

    plsc.subcore_barrier()

    @pl.loop(0, RPT // (2 * EC))
    def _(q):
        r0 = s * RPT + 2 * q * EC
        r1 = r0 + EC
        pltpu.sync_copy(accsh.at[pl.ds(r0, EC)], buf0)
        pltpu.async_copy(buf0, acc_hbm.at[c, pl.ds(r0, EC)], sem0)
        pltpu.sync_copy(accsh.at[pl.ds(r1, EC)], buf1)
        pltpu.async_copy(buf1, acc_hbm.at[c, pl.ds(r1, EC)], sem1)
        pltpu.make_async_copy(buf0, acc_hbm.at[c, pl.ds(r0, EC)], sem0).wait()
        pltpu.make_async_copy(buf1, acc_hbm.at[c, pl.ds(r1, EC)], sem1).wait()


def _dis_block(deg_ref):
    d = deg_ref[0, :, 0:1] + deg_ref[1, :, 0:1] + 1.0
    return lax.rsqrt(d)


def _tc_first_body(deg_ref, h_ref, w_ref, out_ref):
    dis = _dis_block(deg_ref)
    out_ref[...] = dis * jnp.dot(h_ref[...], w_ref[...],
                                 preferred_element_type=jnp.float32)


def _tc_mid_body(deg_ref, acc_ref, g_ref, b_ref, w_ref, out_ref):
    dis = _dis_block(deg_ref)
    h = jax.nn.relu(dis * (acc_ref[0] + acc_ref[1] + g_ref[...]) + b_ref[...])
    out_ref[...] = dis * jnp.dot(h, w_ref[...],
                                 preferred_element_type=jnp.float32)


def _tc_pool_body(deg_ref, acc_ref, g_ref, b_ref, bv_ref,
                  fc1w_ref, fc1b_ref, fc2w_ref, fc2b_ref,
                  out_ref, sacc, cacc):
    pi = pl.program_id(0)

    @pl.when(pi == 0)
    def _():
        sacc[...] = jnp.zeros_like(sacc)
        cacc[...] = jnp.zeros_like(cacc)

    dis = _dis_block(deg_ref)
    h = jax.nn.relu(dis * (acc_ref[0] + acc_ref[1] + g_ref[...]) + b_ref[...])
    bv = bv_ref[0]
    ohT = (lax.broadcasted_iota(jnp.int32, (NG, PB), 0) == bv
           ).astype(jnp.float32)
    sacc[...] += jnp.dot(ohT, h, preferred_element_type=jnp.float32)
    cacc[...] += jnp.sum(ohT, axis=1, keepdims=True)

    @pl.when(pi == (N_NODES // PB) - 1)
    def _():
        mean = sacc[...] / (cacc[...] + 1e-6)
        o1 = jax.nn.relu(jnp.dot(mean, fc1w_ref[...],
                                 preferred_element_type=jnp.float32)
                         + fc1b_ref[...])
        o2 = jnp.dot(o1, fc2w_ref[...],
                     preferred_element_type=jnp.float32) + fc2b_ref[...]
        out_ref[...] = o2[:, 0:1]


def _tc_first(degacc, h0, W1):
    return pl.pallas_call(
        _tc_first_body,
        grid=(N_PAD // TB,),
        in_specs=[
            pl.BlockSpec((NC, TB, H), lambda i: (0, i, 0)),
            pl.BlockSpec((TB, H), lambda i: (i, 0)),
            pl.BlockSpec((H, H), lambda i: (0, 0)),
        ],
        out_specs=pl.BlockSpec((TB, H), lambda i: (i, 0)),
        out_shape=jax.ShapeDtypeStruct((N_PAD, H), jnp.float32),
    )(degacc, h0, W1)


def _tc_mid(degacc, acc, g, br, Wn):
    return pl.pallas_call(
        _tc_mid_body,
        grid=(N_PAD // TB,),
        in_specs=[
            pl.BlockSpec((NC, TB, H), lambda i: (0, i, 0)),
            pl.BlockSpec((NC, TB, H), lambda i: (0, i, 0)),
            pl.BlockSpec((TB, H), lambda i: (i, 0)),
            pl.BlockSpec((1, H), lambda i: (0, 0)),
            pl.BlockSpec((H, H), lambda i: (0, 0)),
        ],
        out_specs=pl.BlockSpec((TB, H), lambda i: (i, 0)),
        out_shape=jax.ShapeDtypeStruct((N_PAD, H), jnp.float32),
    )(degacc, acc, g, br, Wn)


def _tc_pool(degacc, acc, g, br, batch_r, fc1p, fc1bp, fc2p, fc2bp):
    return pl.pallas_call(
        _tc_pool_body,
        grid=(N_NODES // PB,),
        in_specs=[
            pl.BlockSpec((NC, PB, H), lambda i: (0, i, 0)),
            pl.BlockSpec((NC, PB, H), lambda i: (0, i, 0)),
            pl.BlockSpec((PB, H), lambda i: (i, 0)),
            pl.BlockSpec((1, H), lambda i: (0, 0)),
            pl.BlockSpec((1, 1, PB), lambda i: (i, 0, 0)),
            pl.BlockSpec((H, H), lambda i: (0, 0)),
            pl.BlockSpec((1, H), lambda i: (0, 0)),
            pl.BlockSpec((H, H), lambda i: (0, 0)),
            pl.BlockSpec((1, H), lambda i: (0, 0)),
        ],
        out_specs=pl.BlockSpec((NG, 1), lambda i: (0, 0)),
        out_shape=jax.ShapeDtypeStruct((NG, 1), jnp.float32),
        scratch_shapes=[
            pltpu.VMEM((NG, H), jnp.float32),
            pltpu.VMEM((NG, 1), jnp.float32),
        ],
    )(degacc, acc, g, br, batch_r, fc1p, fc1bp, fc2p, fc2bp)


@jax.jit
def _impl(x, edge_index, batch_vec, emb, W1, b1, W2, b2, W3, b3,
          fc1_w, fc1_b, fc2_w, fc2_b):
    f32 = jnp.float32
    i32 = jnp.int32
    x_pad = jnp.concatenate(
        [x.astype(i32), jnp.zeros((N_PAD - N_NODES,), i32)])
    row = edge_index[0].astype(i32)
    col = edge_index[1].astype(i32)
    row_pad = jnp.concatenate(
        [row, jnp.full((E_PAD - E,), N_NODES, i32)]).reshape(NW * CPT, CH)
    col_pad = jnp.concatenate(
        [col, jnp.zeros((E_PAD - E,), i32)]).reshape(NW * CPT, CH)
    row_pad_e = row_pad.reshape(NW * ECT, EC)
    col_pad_e = col_pad.reshape(NW * ECT, EC)
    znode = jnp.concatenate(
        [jnp.zeros((N_PAD, H), f32), jnp.ones((CH, H), f32)])
    batch_r = batch_vec.astype(i32).reshape(N_NODES // PB, 1, PB)
    b1r = b1.reshape(1, H)
    b2r = b2.reshape(1, H)
    b3r = b3.reshape(1, H)
    fc1p = jnp.zeros((H, H), f32).at[:, :32].set(fc1_w)
    fc1bp = jnp.zeros((1, H), f32).at[0, :32].set(fc1_b)
    fc2p = jnp.zeros((H, H), f32).at[:32, 0:1].set(fc2_w)
    fc2bp = jnp.full((1, H), fc2_b[0], f32)

    h0, degacc = _sc_gather_deg(emb, x_pad, row_pad, znode)
    g1 = _tc_first(degacc, h0, W1)
    acc1 = _sc_edge_pass(g1, col_pad_e, row_pad_e, znode)
    g2 = _tc_mid(degacc, acc1, g1, b1r, W2)
    acc2 = _sc_edge_pass(g2, col_pad_e, row_pad_e, znode)
    g3 = _tc_mid(degacc, acc2, g2, b2r, W3)
    acc3 = _sc_edge_pass(g3, col_pad_e, row_pad_e, znode)
    return _tc_pool(degacc, acc3, g3, b3r, batch_r, fc1p, fc1bp, fc2p, fc2bp)


def kernel(x, edge_index, batch_vec, emb, W1, b1, W2, b2, W3, b3,
           fc1_w, fc1_b, fc2_w, fc2_b):
    return _impl(x, edge_index, batch_vec, emb, W1, b1, W2, b2, W3, b3,
                 fc1_w, fc1_b, fc2_w, fc2_b)
